# baseline (speedup 1.0000x reference)
import dataclasses
import functools

import jax
import jax.numpy as jnp
from jax import lax
from jax.experimental import pallas as pl
from jax.experimental.pallas import tpu as pltpu
from jax.experimental.pallas import tpu_sc as plsc

L = 16
W = 128


@functools.lru_cache(maxsize=None)
def _make_gather(Vt, D, N, Vf, H):
    NW = N // W
    mesh = plsc.VectorSubcoreMesh(core_axis_name="c", subcore_axis_name="s")
    cp = pltpu.CompilerParams(use_tc_tiling_on_sc=False)
    if "needs_layout_passes" in pltpu.CompilerParams.__dataclass_fields__:
        cp = dataclasses.replace(cp, needs_layout_passes=False)
    NWORK = 32
    TW = NW // NWORK

    @functools.partial(
        pl.kernel,
        out_type=jax.ShapeDtypeStruct((NW * 4096,), jnp.float32),
        mesh=mesh,
        compiler_params=cp,
        scratch_types=[
            pltpu.VMEM((Vf, D), jnp.float32),
            pltpu.VMEM((4, W), jnp.int32),
            pltpu.VMEM((2, W), jnp.int32),
            pltpu.VMEM((2, W, D), jnp.float32),
            pltpu.VMEM((2, 4096), jnp.float32),
            pltpu.SemaphoreType.DMA((4,)),
            pltpu.SemaphoreType.DMA((2,)),
            pltpu.SemaphoreType.DMA((2,)),
        ],
    )
    def gather_kernel(table_hbm, frozen_hbm, idx_hbm, out_hbm, frozen_v,
                      ibuf, cbuf, rbuf, tbuf, isem, gsem, osem):
        pltpu.sync_copy(frozen_hbm, frozen_v)
        wid = lax.axis_index("s") * 2 + lax.axis_index("c")
        w0 = wid * TW

        def idx_src(w):
            return idx_hbm.at[0, pl.ds((w0 + w) * W, W)]

        def out_dst(w, cb):
            gw = w0 + w
            h = gw // 128
            bb = gw % 128
            r0 = ((h * 4 + cb) * 128 + bb) * 1024
            return out_hbm.at[pl.ds(r0, 1024)]

        def fire_idx(w, ring):
            pltpu.async_copy(idx_src(w), ibuf.at[ring], isem.at[ring])

        def front(w, ring, par):
            pltpu.make_async_copy(idx_src(w), ibuf.at[ring],
                                  isem.at[ring]).wait()

            def clamp_step(k, _):
                v = ibuf[ring, pl.ds(k * L, L)]
                cbuf[par, pl.ds(k * L, L)] = jnp.minimum(v, Vt - 1)
                return 0

            lax.fori_loop(0, W // L, clamp_step, 0)
            pltpu.async_copy(table_hbm.at[cbuf.at[par]], rbuf.at[par],
                             gsem.at[par])

        def back(w, ring, par, wait_out):
            pltpu.make_async_copy(table_hbm.at[cbuf.at[par]], rbuf.at[par],
                                  gsem.at[par]).wait()

            def scan_step(k, mx):
                return jnp.maximum(mx, ibuf[ring, pl.ds(k * L, L)])

            mx = lax.fori_loop(0, W // L, scan_step,
                               jnp.zeros((L,), jnp.int32))

            @pl.when(jnp.max(mx) >= Vt)
            def _fixup():
                def group(k, _):
                    v = ibuf[ring, pl.ds(k * L, L)]
                    msk = v >= Vt
                    fr = jnp.clip(v - Vt, 0, Vf - 1)
                    rows = lax.iota(jnp.int32, L) + k * L

                    def col(c, _):
                        cvec = jnp.zeros((L,), jnp.int32) + c
                        vals = plsc.load_gather(frozen_v, [fr, cvec], mask=msk)
                        plsc.store_scatter(rbuf.at[par], [rows, cvec], vals,
                                           mask=msk)
                        return 0

                    return lax.fori_loop(0, D, col, 0)

                lax.fori_loop(0, W // L, group, 0)

            @pl.when(wait_out)
            def _drain_out():
                for cb in range(4):
                    pltpu.make_async_copy(tbuf.at[par, pl.ds(cb * 1024, 1024)],
                                          out_dst(w - 2, cb),
                                          osem.at[par]).wait()

            bvec = lax.iota(jnp.int32, L)

            def tr_col(c, _):
                def tr_blk(bb, _):
                    vals = plsc.load_gather(
                        rbuf.at[par],
                        [bvec + bb * L, jnp.zeros((L,), jnp.int32) + c])
                    off = (c // 8) * 1024 + (c % 8) * 128 + bb * L
                    tbuf[par, pl.ds(off, L)] = vals
                    return 0

                return lax.fori_loop(0, W // L, tr_blk, 0)

            lax.fori_loop(0, D, tr_col, 0)
            for cb in range(4):
                pltpu.async_copy(tbuf.at[par, pl.ds(cb * 1024, 1024)],
                                 out_dst(w, cb), osem.at[par])

        fire_idx(0, 0)
        fire_idx(1, 1)

        @pl.loop(0, TW + 4, step=4)
        def _(t):
            for b in range(4):
                w = t + b

                @pl.when(w < TW)
                def _f(w=w, b=b):
                    front(w, b, b % 2)

                @pl.when(jnp.logical_and(w + 2 < TW, w + 2 >= 2))
                def _p(w=w, b=b):
                    fire_idx(w + 2, (b + 2) % 4)

                wm1 = w - 1

                @pl.when(jnp.logical_and(wm1 >= 0, wm1 < TW))
                def _b(wm1=wm1, b=b):
                    back(wm1, (b + 3) % 4, (b + 1) % 2, wm1 >= 2)

        for last in (TW - 2, TW - 1):
            par = last % 2
            for cb in range(4):
                pltpu.make_async_copy(tbuf.at[par, pl.ds(cb * 1024, 1024)],
                                      out_dst(last, cb), osem.at[par]).wait()

    return gather_kernel


@jax.jit
def kernel(x, trainable_embedding, frozen_embedding):
    B, H = x.shape
    Vt, D = trainable_embedding.shape
    Vf = frozen_embedding.shape[0]
    N = B * H
    idx = x.T.reshape(1, N).astype(jnp.int32)
    tiles = _make_gather(Vt, D, N, Vf, H)(
        trainable_embedding, frozen_embedding, idx)
    out = (tiles.reshape(H, 4, B // 128, 8, 128)
           .transpose(2, 4, 0, 1, 3)
           .reshape(B, H, D))
    return out


# speedup vs baseline: 1.0824x; 1.0824x over previous
import dataclasses
import functools

import jax
import jax.numpy as jnp
from jax import lax
from jax.experimental import pallas as pl
from jax.experimental.pallas import tpu as pltpu
from jax.experimental.pallas import tpu_sc as plsc

L = 16
W = 128


@functools.lru_cache(maxsize=None)
def _make_gather(Vt, D, N, Vf, H):
    NW = N // W
    mesh = plsc.VectorSubcoreMesh(core_axis_name="c", subcore_axis_name="s")
    cp = pltpu.CompilerParams(use_tc_tiling_on_sc=False)
    if "needs_layout_passes" in pltpu.CompilerParams.__dataclass_fields__:
        cp = dataclasses.replace(cp, needs_layout_passes=False)
    NWORK = 32
    TW = NW // NWORK

    @functools.partial(
        pl.kernel,
        out_type=jax.ShapeDtypeStruct((NW * 4096,), jnp.float32),
        mesh=mesh,
        compiler_params=cp,
        scratch_types=[
            pltpu.VMEM((Vf, D), jnp.float32),
            pltpu.VMEM((4, W), jnp.int32),
            pltpu.VMEM((2, W), jnp.int32),
            pltpu.VMEM((2, W, D), jnp.float32),
            pltpu.VMEM((2, 4096), jnp.float32),
            pltpu.SemaphoreType.DMA((4,)),
            pltpu.SemaphoreType.DMA((2,)),
            pltpu.SemaphoreType.DMA((2,)),
        ],
    )
    def gather_kernel(table_hbm, frozen_hbm, idx_hbm, out_hbm, frozen_v,
                      ibuf, cbuf, rbuf, tbuf, isem, gsem, osem):
        pltpu.sync_copy(frozen_hbm, frozen_v)
        wid = lax.axis_index("s") * 2 + lax.axis_index("c")
        w0 = wid * TW

        def idx_src(w):
            return idx_hbm.at[0, pl.ds((w0 + w) * W, W)]

        def out_dst(w, cb):
            gw = w0 + w
            h = gw // 128
            bb = gw % 128
            r0 = ((h * 4 + cb) * 128 + bb) * 1024
            return out_hbm.at[pl.ds(r0, 1024)]

        def fire_idx(w, ring):
            pltpu.async_copy(idx_src(w), ibuf.at[ring], isem.at[ring])

        def front(w, ring, par):
            pltpu.make_async_copy(idx_src(w), ibuf.at[ring],
                                  isem.at[ring]).wait()

            def clamp_step(k, _):
                v = ibuf[ring, pl.ds(k * L, L)]
                cbuf[par, pl.ds(k * L, L)] = jnp.minimum(v, Vt - 1)
                return 0

            lax.fori_loop(0, W // L, clamp_step, 0)
            pltpu.async_copy(table_hbm.at[cbuf.at[par]], rbuf.at[par],
                             gsem.at[par])

        def back(w, ring, par, wait_out):
            pltpu.make_async_copy(table_hbm.at[cbuf.at[par]], rbuf.at[par],
                                  gsem.at[par]).wait()

            def scan_step(k, mx):
                return jnp.maximum(mx, ibuf[ring, pl.ds(k * L, L)])

            mx = lax.fori_loop(0, W // L, scan_step,
                               jnp.zeros((L,), jnp.int32))

            @pl.when(jnp.max(mx) >= Vt)
            def _fixup():
                def group(k, _):
                    v = ibuf[ring, pl.ds(k * L, L)]
                    msk = v >= Vt
                    fr = jnp.clip(v - Vt, 0, Vf - 1)
                    rows = lax.iota(jnp.int32, L) + k * L

                    def col(c, _):
                        cvec = jnp.zeros((L,), jnp.int32) + c
                        vals = plsc.load_gather(frozen_v, [fr, cvec], mask=msk)
                        plsc.store_scatter(rbuf.at[par], [rows, cvec], vals,
                                           mask=msk)
                        return 0

                    return lax.fori_loop(0, D, col, 0)

                lax.fori_loop(0, W // L, group, 0)

            @pl.when(wait_out)
            def _drain_out():
                for cb in range(4):
                    pltpu.make_async_copy(tbuf.at[par, pl.ds(cb * 1024, 1024)],
                                          out_dst(w - 2, cb),
                                          osem.at[par]).wait()

            bvecs = [lax.iota(jnp.int32, L) + bb * L for bb in range(W // L)]

            def tr_col(c, _):
                cvec = jnp.zeros((L,), jnp.int32) + c
                coff = (c // 8) * 1024 + (c % 8) * 128
                for bb in range(W // L):
                    vals = plsc.load_gather(rbuf.at[par], [bvecs[bb], cvec])
                    tbuf[par, pl.ds(coff + bb * L, L)] = vals
                return 0

            lax.fori_loop(0, D, tr_col, 0)
            for cb in range(4):
                pltpu.async_copy(tbuf.at[par, pl.ds(cb * 1024, 1024)],
                                 out_dst(w, cb), osem.at[par])

        fire_idx(0, 0)
        fire_idx(1, 1)

        @pl.loop(0, TW + 4, step=4)
        def _(t):
            for b in range(4):
                w = t + b

                @pl.when(w < TW)
                def _f(w=w, b=b):
                    front(w, b, b % 2)

                @pl.when(jnp.logical_and(w + 2 < TW, w + 2 >= 2))
                def _p(w=w, b=b):
                    fire_idx(w + 2, (b + 2) % 4)

                wm1 = w - 1

                @pl.when(jnp.logical_and(wm1 >= 0, wm1 < TW))
                def _b(wm1=wm1, b=b):
                    back(wm1, (b + 3) % 4, (b + 1) % 2, wm1 >= 2)

        for last in (TW - 2, TW - 1):
            par = last % 2
            for cb in range(4):
                pltpu.make_async_copy(tbuf.at[par, pl.ds(cb * 1024, 1024)],
                                      out_dst(last, cb), osem.at[par]).wait()

    return gather_kernel


@jax.jit
def kernel(x, trainable_embedding, frozen_embedding):
    B, H = x.shape
    Vt, D = trainable_embedding.shape
    Vf = frozen_embedding.shape[0]
    N = B * H
    idx = x.T.reshape(1, N).astype(jnp.int32)
    tiles = _make_gather(Vt, D, N, Vf, H)(
        trainable_embedding, frozen_embedding, idx)
    out = (tiles.reshape(H, 4, B // 128, 8, 128)
           .transpose(2, 4, 0, 1, 3)
           .reshape(B, H, D))
    return out
